# baseline (speedup 1.0000x reference)
import functools

import jax
import jax.numpy as jnp
from jax import lax
from jax.experimental import pallas as pl
from jax.experimental.pallas import tpu as pltpu
from jax.experimental.pallas import tpu_sc as plsc

N = 100000
D = 128
WINDOW = 400
NWIN = N // WINDOW
NUM_SUBCORES = 16


def _sc_lookup(chi, emb_weight):
    mesh = plsc.VectorSubcoreMesh(core_axis_name="c", subcore_axis_name="s")
    chi3d = chi.reshape(NWIN, 1, WINDOW)
    table8 = jnp.zeros((8, D), emb_weight.dtype).at[:3].set(emb_weight)

    @functools.partial(
        pl.kernel,
        mesh=mesh,
        out_type=jax.ShapeDtypeStruct((N, D), jnp.float32),
        scratch_types=[
            pltpu.VMEM_SHARED((8 * NUM_SUBCORES, D), jnp.float32),
            pltpu.VMEM((WINDOW,), jnp.int32),
        ],
    )
    def k(table_hbm, idx_hbm, out_hbm, table_sh, idx_adj):
        sid = lax.axis_index("s")
        for s in range(NUM_SUBCORES):
            @pl.when(sid == s)
            def _():
                pltpu.sync_copy(table_hbm, table_sh.at[pl.ds(8 * s, 8)])

        bias = sid * 8

        def body(i_vmem, o_vmem):
            @pl.loop(0, WINDOW, step=16)
            def _(c):
                idx_adj[pl.ds(c, 16)] = i_vmem[0, 0, pl.ds(c, 16)] + bias

            pltpu.sync_copy(table_sh.at[idx_adj], o_vmem)

        pltpu.emit_pipeline(
            body,
            grid=(NWIN,),
            in_specs=[pl.BlockSpec((1, 1, WINDOW), index_map=lambda i: (i, 0, 0))],
            out_specs=[pl.BlockSpec((WINDOW, D), index_map=lambda i: (i, 0))],
            core_axis_name=("c", "s"),
            dimension_semantics=(pltpu.PARALLEL,),
        )(idx_hbm, out_hbm)

    return k(table8, chi3d)


def kernel(chi, emb_weight):
    chi = chi.astype(jnp.int32)
    emb_weight = emb_weight.astype(jnp.float32)
    return _sc_lookup(chi, emb_weight)


# speedup vs baseline: 1.0735x; 1.0735x over previous
import functools

import jax
import jax.numpy as jnp
from jax import lax
from jax.experimental import pallas as pl
from jax.experimental.pallas import tpu as pltpu
from jax.experimental.pallas import tpu_sc as plsc

N = 100000
D = 128
WINDOW = 200
NWIN = N // WINDOW


def _sc_lookup(chi, emb_weight):
    mesh = plsc.VectorSubcoreMesh(core_axis_name="c", subcore_axis_name="s")
    chi3d = chi.reshape(NWIN, 1, WINDOW)

    @functools.partial(
        pl.kernel,
        mesh=mesh,
        out_type=jax.ShapeDtypeStruct((N, D), jnp.float32),
        scratch_types=[
            pltpu.VMEM_SHARED((3, D), jnp.float32),
        ],
    )
    def k(table_hbm, idx_hbm, out_hbm, table_sh):
        @pl.when(lax.axis_index("s") == 0)
        def _():
            pltpu.sync_copy(table_hbm, table_sh)

        plsc.subcore_barrier()

        def body(i_vmem, o_vmem):
            pltpu.sync_copy(table_sh.at[i_vmem.at[0, 0]], o_vmem)

        pltpu.emit_pipeline(
            body,
            grid=(NWIN,),
            in_specs=[pl.BlockSpec((1, 1, WINDOW), index_map=lambda i: (i, 0, 0))],
            out_specs=[pl.BlockSpec((WINDOW, D), index_map=lambda i: (i, 0))],
            core_axis_name=("c", "s"),
            dimension_semantics=(pltpu.PARALLEL,),
        )(idx_hbm, out_hbm)

    return k(emb_weight, chi3d)


def kernel(chi, emb_weight):
    chi = chi.astype(jnp.int32)
    emb_weight = emb_weight.astype(jnp.float32)
    return _sc_lookup(chi, emb_weight)
